# Initial kernel scaffold; baseline (speedup 1.0000x reference)
import jax
import jax.numpy as jnp
from jax.experimental import pallas as pl


def kernel(h_index, r_index, t_index, hidden_states, rel_hidden_states, edge_index, score_text_embs, all_index, rel_table, lin_W, lin_b, mlp_W1, mlp_b1, mlp_W2, mlp_b2, pre_W, pre_b, post_W, post_b):
    raise NotImplementedError("write your pallas kernel here")


# speedup vs baseline: 9.5858x; 9.5858x over previous
import jax
import jax.numpy as jnp
from jax.experimental import pallas as pl

_D = 128
_NUM_REL = 16
_NUM_LAYERS = 3
_NODE_RATIO = 0.1
_DELTA = 1.0


def _score_body(hid_ref, rel_ref, lwh_ref, lwr_ref, lb_ref, w1_ref, b1_ref,
                w2_ref, b2_ref, out_ref):
    hid = hid_ref[...]
    heur = (jnp.dot(hid, lwh_ref[...], preferred_element_type=jnp.float32)
            + jnp.dot(rel_ref[...], lwr_ref[...],
                      preferred_element_type=jnp.float32)
            + lb_ref[...])
    x = heur * hid
    y = jax.nn.relu(jnp.dot(x, w1_ref[...], preferred_element_type=jnp.float32)
                    + b1_ref[...])
    out_ref[...] = jnp.sum(y * w2_ref[...], axis=1, keepdims=True) + b2_ref[...]


def _score_pallas(hid, rel, lin_W, lin_b, W1, b1, W2, b2, block_rows):
    rows = hid.shape[0]
    grid = rows // block_rows
    rel_bcast = rel.shape[0] == 1
    rel_spec = (pl.BlockSpec((1, _D), lambda i: (0, 0)) if rel_bcast
                else pl.BlockSpec((block_rows, _D), lambda i: (i, 0)))
    out = pl.pallas_call(
        _score_body,
        grid=(grid,),
        in_specs=[
            pl.BlockSpec((block_rows, _D), lambda i: (i, 0)),
            rel_spec,
            pl.BlockSpec((_D, _D), lambda i: (0, 0)),
            pl.BlockSpec((_D, _D), lambda i: (0, 0)),
            pl.BlockSpec((1, _D), lambda i: (0, 0)),
            pl.BlockSpec((_D, _D), lambda i: (0, 0)),
            pl.BlockSpec((1, _D), lambda i: (0, 0)),
            pl.BlockSpec((1, _D), lambda i: (0, 0)),
            pl.BlockSpec((1, 1), lambda i: (0, 0)),
        ],
        out_specs=pl.BlockSpec((block_rows, 1), lambda i: (i, 0)),
        out_shape=jax.ShapeDtypeStruct((rows, 1), jnp.float32),
    )(hid, rel, lin_W[:_D], lin_W[_D:], lin_b[None, :], W1, b1[None, :],
      W2[:, 0][None, :], b2[None, :])
    return out[:, 0]


def _update_body(hid_ref, scaled_ref, w_ref, b_ref, out_ref):
    acc = jnp.dot(hid_ref[...], w_ref[:_D], preferred_element_type=jnp.float32)
    acc += jnp.dot(scaled_ref[...], w_ref[_D:],
                   preferred_element_type=jnp.float32)
    out_ref[...] = acc + b_ref[...]


def _update_pallas(hid_sel, scaled_row, postW, post_b):
    rows = hid_sel.shape[0]
    return pl.pallas_call(
        _update_body,
        grid=(1,),
        in_specs=[
            pl.BlockSpec((rows, _D), lambda i: (0, 0)),
            pl.BlockSpec((1, 12 * _D), lambda i: (0, 0)),
            pl.BlockSpec((13 * _D, _D), lambda i: (0, 0)),
            pl.BlockSpec((1, _D), lambda i: (0, 0)),
        ],
        out_specs=pl.BlockSpec((rows, _D), lambda i: (0, 0)),
        out_shape=jax.ShapeDtypeStruct((rows, _D), jnp.float32),
    )(hid_sel, scaled_row, postW, post_b[None, :])


def kernel(h_index, r_index, t_index, hidden_states, rel_hidden_states,
           edge_index, score_text_embs, all_index, rel_table, lin_W, lin_b,
           mlp_W1, mlp_b1, mlp_W2, mlp_b2, pre_W, pre_b, post_W, post_b):
    del rel_hidden_states, all_index, pre_W, pre_b
    Bn, K = h_index.shape
    Nn = score_text_embs.shape[0]
    kk = max(1, int(_NODE_RATIO * Nn))

    is_t_neg = jnp.all(h_index == h_index[:, :1], axis=-1, keepdims=True)
    h = jnp.where(is_t_neg, h_index, t_index)
    t = jnp.where(is_t_neg, t_index, h_index)
    r = jnp.where(is_t_neg, r_index, r_index + _NUM_REL)
    head_orig = h[:, 0]
    rel_emb = rel_table[r[:, 0]]

    appears = jnp.zeros((Nn,), dtype=bool)
    appears = appears.at[edge_index[0]].set(True)
    appears = appears.at[edge_index[1]].set(True)

    hid0 = score_text_embs.at[head_orig[0]].set(hidden_states[0])
    hid1 = score_text_embs.at[head_orig[1]].set(hidden_states[1])

    def _score_small(hid2, rel2):
        hh = jnp.concatenate([hid2, rel2], axis=-1)
        heur = hh @ lin_W + lin_b
        x = heur * hid2
        y = jax.nn.relu(x @ mlp_W1 + mlp_b1) @ mlp_W2 + mlp_b2
        return y[..., 0]

    background = _score_small(jnp.zeros_like(rel_emb), rel_emb)
    head_score = _score_small(hidden_states, rel_emb)
    score0 = jnp.full((Nn,), background[0]).at[head_orig[0]].set(head_score[0])

    std_c = jnp.sqrt(jnp.float32(0.0) + 1e-5)
    lg2 = jnp.log(jnp.float32(2.0))
    zeros_blk = jnp.zeros((3 * _D,), jnp.float32)
    std_blk = jnp.full((_D,), std_c, jnp.float32)
    scaled_row = jnp.concatenate([
        zeros_blk, std_blk,
        zeros_blk, std_blk * (lg2 / _DELTA),
        zeros_blk, std_blk * (_DELTA / lg2),
    ])[None, :]

    base_done = False
    for li in range(_NUM_LAYERS):
        _, idx = jax.lax.top_k(score0, kk)
        hid_sel = hid0[idx]
        nh = _update_pallas(hid_sel, scaled_row, post_W[li], post_b[li])
        ok = appears[idx][:, None]
        hid0 = hid0.at[idx].add(jnp.where(ok, nh, 0.0))
        if li < _NUM_LAYERS - 1:
            if not base_done:
                score0 = _score_pallas(hid0, rel_emb[0:1], lin_W, lin_b,
                                       mlp_W1, mlp_b1, mlp_W2, mlp_b2,
                                       block_rows=2000)
                base_done = True
            else:
                snew = _score_pallas(hid0[idx], rel_emb[0:1], lin_W, lin_b,
                                     mlp_W1, mlp_b1, mlp_W2, mlp_b2,
                                     block_rows=kk)
                score0 = score0.at[idx].set(snew)

    t_flat = t.reshape(-1)
    hid_t = jnp.where((jnp.arange(Bn * K) < K)[:, None],
                      hid0[t_flat], hid1[t_flat])
    rel_t = jnp.repeat(rel_emb, K, axis=0)
    sc_t = _score_pallas(hid_t, rel_t, lin_W, lin_b, mlp_W1, mlp_b1,
                         mlp_W2, mlp_b2, block_rows=Bn * K)
    return sc_t.reshape(Bn, K)


# speedup vs baseline: 10.0457x; 1.0480x over previous
import functools
import jax
import jax.numpy as jnp
from jax import lax
from jax.experimental import pallas as pl
from jax.experimental.pallas import tpu as pltpu

_D = 128
_NUM_REL = 16
_NUM_LAYERS = 3
_NODE_RATIO = 0.1
_DELTA = 1.0


def _score_rows(hid, rel, lwh_ref, lwr_ref, lb_ref, w1_ref, b1_ref, w2_ref,
                b2_ref):
    heur = (jnp.dot(hid, lwh_ref[...], preferred_element_type=jnp.float32)
            + jnp.dot(rel, lwr_ref[...], preferred_element_type=jnp.float32)
            + lb_ref[...])
    x = heur * hid
    y = jax.nn.relu(jnp.dot(x, w1_ref[...], preferred_element_type=jnp.float32)
                    + b1_ref[...])
    return jnp.sum(y * w2_ref[...], axis=1, keepdims=True) + b2_ref[...]


def _layer_body(idx_ref, hid_ref, sc_ref, app_ref, srow_ref, w_ref, b_ref,
                rel_ref, lwh_ref, lwr_ref, lb_ref, w1_ref, b1_ref, w2_ref,
                b2_ref, hid_out, sc_out, rows_scr, *, kk, full_score):
    def gbody(j, c):
        r = idx_ref[j]
        rows_scr[pl.ds(j, 1), :] = hid_ref[pl.ds(r, 1), :]
        return c

    lax.fori_loop(0, kk, gbody, 0)
    hs = rows_scr[...]
    nh = (jnp.dot(hs, w_ref[:_D], preferred_element_type=jnp.float32)
          + jnp.dot(srow_ref[...], w_ref[_D:],
                    preferred_element_type=jnp.float32)
          + b_ref[...])
    newr = hs + app_ref[...] * nh
    rows_scr[...] = newr

    hid_out[...] = hid_ref[...]

    def sbody(j, c):
        r = idx_ref[j]
        hid_out[pl.ds(r, 1), :] = rows_scr[pl.ds(j, 1), :]
        return c

    lax.fori_loop(0, kk, sbody, 0)

    rel_row = rel_ref[...]
    if full_score:
        sc_out[...] = _score_rows(hid_out[...], rel_row, lwh_ref, lwr_ref,
                                  lb_ref, w1_ref, b1_ref, w2_ref, b2_ref)
    else:
        sc_out[...] = sc_ref[...]
        svals = _score_rows(newr, rel_row, lwh_ref, lwr_ref, lb_ref, w1_ref,
                            b1_ref, w2_ref, b2_ref)
        rows_scr[:, 0:1] = svals

        def scb(j, c):
            r = idx_ref[j]
            sc_out[pl.ds(r, 1), :] = rows_scr[pl.ds(j, 1), 0:1]
            return c

        lax.fori_loop(0, kk, scb, 0)


def _layer_pallas(idx, hid0, sc0, app_sel, scaled_row, postW, post_b, rel_row,
                  lin_W, lin_b, W1, b1, W2, b2, *, full_score):
    Nn = hid0.shape[0]
    kk = idx.shape[0]
    spec = pltpu.PrefetchScalarGridSpec(
        num_scalar_prefetch=1,
        grid=(1,),
        in_specs=[
            pl.BlockSpec((Nn, _D), lambda i, s: (0, 0)),
            pl.BlockSpec((Nn, 1), lambda i, s: (0, 0)),
            pl.BlockSpec((kk, 1), lambda i, s: (0, 0)),
            pl.BlockSpec((1, 12 * _D), lambda i, s: (0, 0)),
            pl.BlockSpec((13 * _D, _D), lambda i, s: (0, 0)),
            pl.BlockSpec((1, _D), lambda i, s: (0, 0)),
            pl.BlockSpec((1, _D), lambda i, s: (0, 0)),
            pl.BlockSpec((_D, _D), lambda i, s: (0, 0)),
            pl.BlockSpec((_D, _D), lambda i, s: (0, 0)),
            pl.BlockSpec((1, _D), lambda i, s: (0, 0)),
            pl.BlockSpec((_D, _D), lambda i, s: (0, 0)),
            pl.BlockSpec((1, _D), lambda i, s: (0, 0)),
            pl.BlockSpec((1, _D), lambda i, s: (0, 0)),
            pl.BlockSpec((1, 1), lambda i, s: (0, 0)),
        ],
        out_specs=[
            pl.BlockSpec((Nn, _D), lambda i, s: (0, 0)),
            pl.BlockSpec((Nn, 1), lambda i, s: (0, 0)),
        ],
        scratch_shapes=[pltpu.VMEM((kk, _D), jnp.float32)],
    )
    body = functools.partial(_layer_body, kk=kk, full_score=full_score)
    hid_new, sc_new = pl.pallas_call(
        body,
        grid_spec=spec,
        out_shape=[
            jax.ShapeDtypeStruct((Nn, _D), jnp.float32),
            jax.ShapeDtypeStruct((Nn, 1), jnp.float32),
        ],
    )(idx, hid0, sc0, app_sel, scaled_row, postW, post_b[None, :], rel_row,
      lin_W[:_D], lin_W[_D:], lin_b[None, :], W1, b1[None, :],
      W2[:, 0][None, :], b2[None, :])
    return hid_new, sc_new


def _score_body(hid_ref, rel_ref, lwh_ref, lwr_ref, lb_ref, w1_ref, b1_ref,
                w2_ref, b2_ref, out_ref):
    out_ref[...] = _score_rows(hid_ref[...], rel_ref[...], lwh_ref, lwr_ref,
                               lb_ref, w1_ref, b1_ref, w2_ref, b2_ref)


def _score_pallas(hid, rel, lin_W, lin_b, W1, b1, W2, b2):
    rows = hid.shape[0]
    out = pl.pallas_call(
        _score_body,
        grid=(1,),
        in_specs=[
            pl.BlockSpec((rows, _D), lambda i: (0, 0)),
            pl.BlockSpec((rows, _D), lambda i: (0, 0)),
            pl.BlockSpec((_D, _D), lambda i: (0, 0)),
            pl.BlockSpec((_D, _D), lambda i: (0, 0)),
            pl.BlockSpec((1, _D), lambda i: (0, 0)),
            pl.BlockSpec((_D, _D), lambda i: (0, 0)),
            pl.BlockSpec((1, _D), lambda i: (0, 0)),
            pl.BlockSpec((1, _D), lambda i: (0, 0)),
            pl.BlockSpec((1, 1), lambda i: (0, 0)),
        ],
        out_specs=pl.BlockSpec((rows, 1), lambda i: (0, 0)),
        out_shape=jax.ShapeDtypeStruct((rows, 1), jnp.float32),
    )(hid, rel, lin_W[:_D], lin_W[_D:], lin_b[None, :], W1, b1[None, :],
      W2[:, 0][None, :], b2[None, :])
    return out[:, 0]


def kernel(h_index, r_index, t_index, hidden_states, rel_hidden_states,
           edge_index, score_text_embs, all_index, rel_table, lin_W, lin_b,
           mlp_W1, mlp_b1, mlp_W2, mlp_b2, pre_W, pre_b, post_W, post_b):
    del rel_hidden_states, all_index, pre_W, pre_b
    Bn, K = h_index.shape
    Nn = score_text_embs.shape[0]
    kk = max(1, int(_NODE_RATIO * Nn))

    is_t_neg = jnp.all(h_index == h_index[:, :1], axis=-1, keepdims=True)
    h = jnp.where(is_t_neg, h_index, t_index)
    t = jnp.where(is_t_neg, t_index, h_index)
    r = jnp.where(is_t_neg, r_index, r_index + _NUM_REL)
    head_orig = h[:, 0]
    rel_emb = rel_table[r[:, 0]]

    appears = jnp.zeros((Nn,), dtype=bool)
    appears = appears.at[edge_index[0]].set(True)
    appears = appears.at[edge_index[1]].set(True)

    hid0 = score_text_embs.at[head_orig[0]].set(hidden_states[0])
    hid1 = score_text_embs.at[head_orig[1]].set(hidden_states[1])

    def _score_small(hid2, rel2):
        hh = jnp.concatenate([hid2, rel2], axis=-1)
        heur = hh @ lin_W + lin_b
        x = heur * hid2
        y = jax.nn.relu(x @ mlp_W1 + mlp_b1) @ mlp_W2 + mlp_b2
        return y[..., 0]

    background = _score_small(jnp.zeros_like(rel_emb), rel_emb)
    head_score = _score_small(hidden_states, rel_emb)
    score0 = jnp.full((Nn,), background[0]).at[head_orig[0]].set(head_score[0])
    sc0 = score0[:, None]

    std_c = jnp.sqrt(jnp.float32(0.0) + 1e-5)
    lg2 = jnp.log(jnp.float32(2.0))
    zeros_blk = jnp.zeros((3 * _D,), jnp.float32)
    std_blk = jnp.full((_D,), std_c, jnp.float32)
    scaled_row = jnp.concatenate([
        zeros_blk, std_blk,
        zeros_blk, std_blk * (lg2 / _DELTA),
        zeros_blk, std_blk * (_DELTA / lg2),
    ])[None, :]

    rel_row = rel_emb[0:1]
    for li in range(_NUM_LAYERS):
        _, idx = jax.lax.top_k(sc0[:, 0], kk)
        app_sel = appears[idx].astype(jnp.float32)[:, None]
        hid0, sc0 = _layer_pallas(idx, hid0, sc0, app_sel, scaled_row,
                                  post_W[li], post_b[li], rel_row, lin_W,
                                  lin_b, mlp_W1, mlp_b1, mlp_W2, mlp_b2,
                                  full_score=(li == 0))

    t_flat = t.reshape(-1)
    hid_t = jnp.where((jnp.arange(Bn * K) < K)[:, None],
                      hid0[t_flat], hid1[t_flat])
    rel_t = jnp.repeat(rel_emb, K, axis=0)
    sc_t = _score_pallas(hid_t, rel_t, lin_W, lin_b, mlp_W1, mlp_b1,
                         mlp_W2, mlp_b2)
    return sc_t.reshape(Bn, K)


# speedup vs baseline: 38.4707x; 3.8296x over previous
import functools
import jax
import jax.numpy as jnp
from jax import lax
from jax.experimental import pallas as pl
from jax.experimental.pallas import tpu as pltpu

_D = 128
_NUM_REL = 16
_NUM_LAYERS = 3
_NODE_RATIO = 0.1
_DELTA = 1.0


def _score_rows(hid, rel, lwh_ref, lwr_ref, lb_ref, w1_ref, b1_ref, w2_ref,
                b2_ref):
    heur = (jnp.dot(hid, lwh_ref[...], preferred_element_type=jnp.float32)
            + jnp.dot(rel, lwr_ref[...], preferred_element_type=jnp.float32)
            + lb_ref[...])
    x = heur * hid
    y = jax.nn.relu(jnp.dot(x, w1_ref[...], preferred_element_type=jnp.float32)
                    + b1_ref[...])
    return jnp.sum(y * w2_ref[...], axis=1, keepdims=True) + b2_ref[...]


def _layer_body(idx_ref, hid_ref, sc_ref, app_ref, srow_ref, w_ref, b_ref,
                rel_ref, lwh_ref, lwr_ref, lb_ref, w1_ref, b1_ref, w2_ref,
                b2_ref, hid_out, sc_out, rows_scr, *, kk, full_score):
    def gbody(j, c):
        r = idx_ref[j]
        rows_scr[pl.ds(j, 1), :] = hid_ref[pl.ds(r, 1), :]
        return c

    lax.fori_loop(0, kk, gbody, 0)
    hs = rows_scr[...]
    nh = (jnp.dot(hs, w_ref[:_D], preferred_element_type=jnp.float32)
          + jnp.dot(srow_ref[...], w_ref[_D:],
                    preferred_element_type=jnp.float32)
          + b_ref[...])
    newr = hs + app_ref[...] * nh
    rows_scr[...] = newr

    hid_out[...] = hid_ref[...]

    def sbody(j, c):
        r = idx_ref[j]
        hid_out[pl.ds(r, 1), :] = rows_scr[pl.ds(j, 1), :]
        return c

    lax.fori_loop(0, kk, sbody, 0)

    rel_row = rel_ref[...]
    if full_score:
        sc_out[...] = _score_rows(hid_out[...], rel_row, lwh_ref, lwr_ref,
                                  lb_ref, w1_ref, b1_ref, w2_ref, b2_ref)
    else:
        sc_out[...] = sc_ref[...]
        svals = _score_rows(newr, rel_row, lwh_ref, lwr_ref, lb_ref, w1_ref,
                            b1_ref, w2_ref, b2_ref)
        rows_scr[:, 0:1] = svals

        def scb(j, c):
            r = idx_ref[j]
            sc_out[pl.ds(r, 1), :] = rows_scr[pl.ds(j, 1), 0:1]
            return c

        lax.fori_loop(0, kk, scb, 0)


def _layer_pallas(idx, hid0, sc0, app_sel, scaled_row, postW, post_b, rel_row,
                  lin_W, lin_b, W1, b1, W2, b2, *, full_score):
    Nn = hid0.shape[0]
    kk = idx.shape[0]
    spec = pltpu.PrefetchScalarGridSpec(
        num_scalar_prefetch=1,
        grid=(1,),
        in_specs=[
            pl.BlockSpec((Nn, _D), lambda i, s: (0, 0)),
            pl.BlockSpec((Nn, 1), lambda i, s: (0, 0)),
            pl.BlockSpec((kk, 1), lambda i, s: (0, 0)),
            pl.BlockSpec((1, 12 * _D), lambda i, s: (0, 0)),
            pl.BlockSpec((13 * _D, _D), lambda i, s: (0, 0)),
            pl.BlockSpec((1, _D), lambda i, s: (0, 0)),
            pl.BlockSpec((1, _D), lambda i, s: (0, 0)),
            pl.BlockSpec((_D, _D), lambda i, s: (0, 0)),
            pl.BlockSpec((_D, _D), lambda i, s: (0, 0)),
            pl.BlockSpec((1, _D), lambda i, s: (0, 0)),
            pl.BlockSpec((_D, _D), lambda i, s: (0, 0)),
            pl.BlockSpec((1, _D), lambda i, s: (0, 0)),
            pl.BlockSpec((1, _D), lambda i, s: (0, 0)),
            pl.BlockSpec((1, 1), lambda i, s: (0, 0)),
        ],
        out_specs=[
            pl.BlockSpec((Nn, _D), lambda i, s: (0, 0)),
            pl.BlockSpec((Nn, 1), lambda i, s: (0, 0)),
        ],
        scratch_shapes=[pltpu.VMEM((kk, _D), jnp.float32)],
    )
    body = functools.partial(_layer_body, kk=kk, full_score=full_score)
    hid_new, sc_new = pl.pallas_call(
        body,
        grid_spec=spec,
        out_shape=[
            jax.ShapeDtypeStruct((Nn, _D), jnp.float32),
            jax.ShapeDtypeStruct((Nn, 1), jnp.float32),
        ],
    )(idx, hid0, sc0, app_sel, scaled_row, postW, post_b[None, :], rel_row,
      lin_W[:_D], lin_W[_D:], lin_b[None, :], W1, b1[None, :],
      W2[:, 0][None, :], b2[None, :])
    return hid_new, sc_new


def _score_body(hid_ref, rel_ref, lwh_ref, lwr_ref, lb_ref, w1_ref, b1_ref,
                w2_ref, b2_ref, out_ref):
    out_ref[...] = _score_rows(hid_ref[...], rel_ref[...], lwh_ref, lwr_ref,
                               lb_ref, w1_ref, b1_ref, w2_ref, b2_ref)


def _score_pallas(hid, rel, lin_W, lin_b, W1, b1, W2, b2):
    rows = hid.shape[0]
    out = pl.pallas_call(
        _score_body,
        grid=(1,),
        in_specs=[
            pl.BlockSpec((rows, _D), lambda i: (0, 0)),
            pl.BlockSpec((rows, _D), lambda i: (0, 0)),
            pl.BlockSpec((_D, _D), lambda i: (0, 0)),
            pl.BlockSpec((_D, _D), lambda i: (0, 0)),
            pl.BlockSpec((1, _D), lambda i: (0, 0)),
            pl.BlockSpec((_D, _D), lambda i: (0, 0)),
            pl.BlockSpec((1, _D), lambda i: (0, 0)),
            pl.BlockSpec((1, _D), lambda i: (0, 0)),
            pl.BlockSpec((1, 1), lambda i: (0, 0)),
        ],
        out_specs=pl.BlockSpec((rows, 1), lambda i: (0, 0)),
        out_shape=jax.ShapeDtypeStruct((rows, 1), jnp.float32),
    )(hid, rel, lin_W[:_D], lin_W[_D:], lin_b[None, :], W1, b1[None, :],
      W2[:, 0][None, :], b2[None, :])
    return out[:, 0]


def kernel(h_index, r_index, t_index, hidden_states, rel_hidden_states,
           edge_index, score_text_embs, all_index, rel_table, lin_W, lin_b,
           mlp_W1, mlp_b1, mlp_W2, mlp_b2, pre_W, pre_b, post_W, post_b):
    del rel_hidden_states, all_index, pre_W, pre_b
    Bn, K = h_index.shape
    Nn = score_text_embs.shape[0]
    kk = max(1, int(_NODE_RATIO * Nn))

    is_t_neg = jnp.all(h_index == h_index[:, :1], axis=-1, keepdims=True)
    h = jnp.where(is_t_neg, h_index, t_index)
    t = jnp.where(is_t_neg, t_index, h_index)
    r = jnp.where(is_t_neg, r_index, r_index + _NUM_REL)
    head_orig = h[:, 0]
    rel_emb = rel_table[r[:, 0]]

    counts = jnp.zeros((Nn,), jnp.float32).at[edge_index.reshape(-1)].add(1.0)
    appears = counts > 0.0

    hid0 = score_text_embs.at[head_orig[0]].set(hidden_states[0])
    hid1 = score_text_embs.at[head_orig[1]].set(hidden_states[1])

    def _score_small(hid2, rel2):
        hh = jnp.concatenate([hid2, rel2], axis=-1)
        heur = hh @ lin_W + lin_b
        x = heur * hid2
        y = jax.nn.relu(x @ mlp_W1 + mlp_b1) @ mlp_W2 + mlp_b2
        return y[..., 0]

    background = _score_small(jnp.zeros_like(rel_emb), rel_emb)
    head_score = _score_small(hidden_states, rel_emb)
    score0 = jnp.full((Nn,), background[0]).at[head_orig[0]].set(head_score[0])
    sc0 = score0[:, None]

    std_c = jnp.sqrt(jnp.float32(0.0) + 1e-5)
    lg2 = jnp.log(jnp.float32(2.0))
    zeros_blk = jnp.zeros((3 * _D,), jnp.float32)
    std_blk = jnp.full((_D,), std_c, jnp.float32)
    scaled_row = jnp.concatenate([
        zeros_blk, std_blk,
        zeros_blk, std_blk * (lg2 / _DELTA),
        zeros_blk, std_blk * (_DELTA / lg2),
    ])[None, :]

    rel_row = rel_emb[0:1]
    for li in range(_NUM_LAYERS):
        _, idx = jax.lax.top_k(sc0[:, 0], kk)
        app_sel = appears[idx].astype(jnp.float32)[:, None]
        hid0, sc0 = _layer_pallas(idx, hid0, sc0, app_sel, scaled_row,
                                  post_W[li], post_b[li], rel_row, lin_W,
                                  lin_b, mlp_W1, mlp_b1, mlp_W2, mlp_b2,
                                  full_score=(li == 0))

    t_flat = t.reshape(-1)
    hid_t = jnp.where((jnp.arange(Bn * K) < K)[:, None],
                      hid0[t_flat], hid1[t_flat])
    rel_t = jnp.repeat(rel_emb, K, axis=0)
    sc_t = _score_pallas(hid_t, rel_t, lin_W, lin_b, mlp_W1, mlp_b1,
                         mlp_W2, mlp_b2)
    return sc_t.reshape(Bn, K)


# speedup vs baseline: 106.9076x; 2.7789x over previous
import functools
import jax
import jax.numpy as jnp
from jax import lax
from jax.experimental import pallas as pl
from jax.experimental.pallas import tpu as pltpu
from jax.experimental.pallas import tpu_sc as plsc

_D = 128
_NUM_REL = 16
_NUM_LAYERS = 3
_NODE_RATIO = 0.1
_DELTA = 1.0


def _score_rows(hid, rel, lwh_ref, lwr_ref, lb_ref, w1_ref, b1_ref, w2_ref,
                b2_ref):
    heur = (jnp.dot(hid, lwh_ref[...], preferred_element_type=jnp.float32)
            + jnp.dot(rel, lwr_ref[...], preferred_element_type=jnp.float32)
            + lb_ref[...])
    x = heur * hid
    y = jax.nn.relu(jnp.dot(x, w1_ref[...], preferred_element_type=jnp.float32)
                    + b1_ref[...])
    return jnp.sum(y * w2_ref[...], axis=1, keepdims=True) + b2_ref[...]


def _layer_body(idx_ref, hid_ref, sc_ref, app_ref, srow_ref, w_ref, b_ref,
                rel_ref, lwh_ref, lwr_ref, lb_ref, w1_ref, b1_ref, w2_ref,
                b2_ref, hid_out, sc_out, rows_scr, *, kk, full_score):
    def gbody(j, c):
        r = idx_ref[j]
        rows_scr[pl.ds(j, 1), :] = hid_ref[pl.ds(r, 1), :]
        return c

    lax.fori_loop(0, kk, gbody, 0)
    hs = rows_scr[...]
    nh = (jnp.dot(hs, w_ref[:_D], preferred_element_type=jnp.float32)
          + jnp.dot(srow_ref[...], w_ref[_D:],
                    preferred_element_type=jnp.float32)
          + b_ref[...])
    newr = hs + app_ref[...] * nh
    rows_scr[...] = newr

    hid_out[...] = hid_ref[...]

    def sbody(j, c):
        r = idx_ref[j]
        hid_out[pl.ds(r, 1), :] = rows_scr[pl.ds(j, 1), :]
        return c

    lax.fori_loop(0, kk, sbody, 0)

    rel_row = rel_ref[...]
    if full_score:
        sc_out[...] = _score_rows(hid_out[...], rel_row, lwh_ref, lwr_ref,
                                  lb_ref, w1_ref, b1_ref, w2_ref, b2_ref)
    else:
        sc_out[...] = sc_ref[...]
        svals = _score_rows(newr, rel_row, lwh_ref, lwr_ref, lb_ref, w1_ref,
                            b1_ref, w2_ref, b2_ref)
        rows_scr[:, 0:1] = svals

        def scb(j, c):
            r = idx_ref[j]
            sc_out[pl.ds(r, 1), :] = rows_scr[pl.ds(j, 1), 0:1]
            return c

        lax.fori_loop(0, kk, scb, 0)


def _layer_pallas(idx, hid0, sc0, app_sel, scaled_row, postW, post_b, rel_row,
                  lin_W, lin_b, W1, b1, W2, b2, *, full_score):
    Nn = hid0.shape[0]
    kk = idx.shape[0]
    spec = pltpu.PrefetchScalarGridSpec(
        num_scalar_prefetch=1,
        grid=(1,),
        in_specs=[
            pl.BlockSpec((Nn, _D), lambda i, s: (0, 0)),
            pl.BlockSpec((Nn, 1), lambda i, s: (0, 0)),
            pl.BlockSpec((kk, 1), lambda i, s: (0, 0)),
            pl.BlockSpec((1, 12 * _D), lambda i, s: (0, 0)),
            pl.BlockSpec((13 * _D, _D), lambda i, s: (0, 0)),
            pl.BlockSpec((1, _D), lambda i, s: (0, 0)),
            pl.BlockSpec((1, _D), lambda i, s: (0, 0)),
            pl.BlockSpec((_D, _D), lambda i, s: (0, 0)),
            pl.BlockSpec((_D, _D), lambda i, s: (0, 0)),
            pl.BlockSpec((1, _D), lambda i, s: (0, 0)),
            pl.BlockSpec((_D, _D), lambda i, s: (0, 0)),
            pl.BlockSpec((1, _D), lambda i, s: (0, 0)),
            pl.BlockSpec((1, _D), lambda i, s: (0, 0)),
            pl.BlockSpec((1, 1), lambda i, s: (0, 0)),
        ],
        out_specs=[
            pl.BlockSpec((Nn, _D), lambda i, s: (0, 0)),
            pl.BlockSpec((Nn, 1), lambda i, s: (0, 0)),
        ],
        scratch_shapes=[pltpu.VMEM((kk, _D), jnp.float32)],
    )
    body = functools.partial(_layer_body, kk=kk, full_score=full_score)
    hid_new, sc_new = pl.pallas_call(
        body,
        grid_spec=spec,
        out_shape=[
            jax.ShapeDtypeStruct((Nn, _D), jnp.float32),
            jax.ShapeDtypeStruct((Nn, 1), jnp.float32),
        ],
    )(idx, hid0, sc0, app_sel, scaled_row, postW, post_b[None, :], rel_row,
      lin_W[:_D], lin_W[_D:], lin_b[None, :], W1, b1[None, :],
      W2[:, 0][None, :], b2[None, :])
    return hid_new, sc_new


def _score_body(hid_ref, rel_ref, lwh_ref, lwr_ref, lb_ref, w1_ref, b1_ref,
                w2_ref, b2_ref, out_ref):
    out_ref[...] = _score_rows(hid_ref[...], rel_ref[...], lwh_ref, lwr_ref,
                               lb_ref, w1_ref, b1_ref, w2_ref, b2_ref)


def _score_pallas(hid, rel, lin_W, lin_b, W1, b1, W2, b2):
    rows = hid.shape[0]
    out = pl.pallas_call(
        _score_body,
        grid=(1,),
        in_specs=[
            pl.BlockSpec((rows, _D), lambda i: (0, 0)),
            pl.BlockSpec((rows, _D), lambda i: (0, 0)),
            pl.BlockSpec((_D, _D), lambda i: (0, 0)),
            pl.BlockSpec((_D, _D), lambda i: (0, 0)),
            pl.BlockSpec((1, _D), lambda i: (0, 0)),
            pl.BlockSpec((_D, _D), lambda i: (0, 0)),
            pl.BlockSpec((1, _D), lambda i: (0, 0)),
            pl.BlockSpec((1, _D), lambda i: (0, 0)),
            pl.BlockSpec((1, 1), lambda i: (0, 0)),
        ],
        out_specs=pl.BlockSpec((rows, 1), lambda i: (0, 0)),
        out_shape=jax.ShapeDtypeStruct((rows, 1), jnp.float32),
    )(hid, rel, lin_W[:_D], lin_W[_D:], lin_b[None, :], W1, b1[None, :],
      W2[:, 0][None, :], b2[None, :])
    return out[:, 0]


def _appears_counts_sc(ids, Nn):
    NC, NS = 2, 16
    total = ids.shape[0]
    chunk = total // (NC * NS)
    mesh = plsc.VectorSubcoreMesh(core_axis_name="c", subcore_axis_name="s")

    @functools.partial(
        pl.kernel, mesh=mesh,
        out_type=jax.ShapeDtypeStruct((NC, Nn), jnp.float32),
        scratch_types=[
            pltpu.VMEM((chunk,), jnp.int32),
            pltpu.VMEM((chunk,), jnp.float32),
            pltpu.VMEM((Nn,), jnp.float32),
            pltpu.VMEM_SHARED((Nn,), jnp.float32),
        ],
    )
    def k(ids_hbm, out_hbm, idx_v, ones_v, zeros_v, shared):
        c = lax.axis_index("c")
        s = lax.axis_index("s")
        base = (c * NS + s) * chunk

        def fill_ones(i, carry):
            ones_v[pl.ds(i * 16, 16)] = jnp.full((16,), 1.0, jnp.float32)
            return carry

        lax.fori_loop(0, chunk // 16, fill_ones, 0)

        @pl.when(s == 0)
        def _():
            def fill_zeros(i, carry):
                zeros_v[pl.ds(i * 16, 16)] = jnp.zeros((16,), jnp.float32)
                return carry

            lax.fori_loop(0, Nn // 16, fill_zeros, 0)
            pltpu.sync_copy(zeros_v, shared)

        pltpu.sync_copy(ids_hbm.at[pl.ds(base, chunk)], idx_v)
        plsc.subcore_barrier()
        pltpu.sync_copy(ones_v, shared.at[idx_v], add=True)
        plsc.subcore_barrier()

        @pl.when(s == 0)
        def _():
            pltpu.sync_copy(shared, out_hbm.at[c])

    return k(ids)


def kernel(h_index, r_index, t_index, hidden_states, rel_hidden_states,
           edge_index, score_text_embs, all_index, rel_table, lin_W, lin_b,
           mlp_W1, mlp_b1, mlp_W2, mlp_b2, pre_W, pre_b, post_W, post_b):
    del rel_hidden_states, all_index, pre_W, pre_b
    Bn, K = h_index.shape
    Nn = score_text_embs.shape[0]
    kk = max(1, int(_NODE_RATIO * Nn))

    is_t_neg = jnp.all(h_index == h_index[:, :1], axis=-1, keepdims=True)
    h = jnp.where(is_t_neg, h_index, t_index)
    t = jnp.where(is_t_neg, t_index, h_index)
    r = jnp.where(is_t_neg, r_index, r_index + _NUM_REL)
    head_orig = h[:, 0]
    rel_emb = rel_table[r[:, 0]]

    counts2 = _appears_counts_sc(edge_index.reshape(-1), Nn)
    appears = (counts2[0] + counts2[1]) > 0.0

    hid0 = score_text_embs.at[head_orig[0]].set(hidden_states[0])
    hid1 = score_text_embs.at[head_orig[1]].set(hidden_states[1])

    def _score_small(hid2, rel2):
        hh = jnp.concatenate([hid2, rel2], axis=-1)
        heur = hh @ lin_W + lin_b
        x = heur * hid2
        y = jax.nn.relu(x @ mlp_W1 + mlp_b1) @ mlp_W2 + mlp_b2
        return y[..., 0]

    background = _score_small(jnp.zeros_like(rel_emb), rel_emb)
    head_score = _score_small(hidden_states, rel_emb)
    score0 = jnp.full((Nn,), background[0]).at[head_orig[0]].set(head_score[0])
    sc0 = score0[:, None]

    std_c = jnp.sqrt(jnp.float32(0.0) + 1e-5)
    lg2 = jnp.log(jnp.float32(2.0))
    zeros_blk = jnp.zeros((3 * _D,), jnp.float32)
    std_blk = jnp.full((_D,), std_c, jnp.float32)
    scaled_row = jnp.concatenate([
        zeros_blk, std_blk,
        zeros_blk, std_blk * (lg2 / _DELTA),
        zeros_blk, std_blk * (_DELTA / lg2),
    ])[None, :]

    rel_row = rel_emb[0:1]
    for li in range(_NUM_LAYERS):
        _, idx = jax.lax.top_k(sc0[:, 0], kk)
        app_sel = appears[idx].astype(jnp.float32)[:, None]
        hid0, sc0 = _layer_pallas(idx, hid0, sc0, app_sel, scaled_row,
                                  post_W[li], post_b[li], rel_row, lin_W,
                                  lin_b, mlp_W1, mlp_b1, mlp_W2, mlp_b2,
                                  full_score=(li == 0))

    t_flat = t.reshape(-1)
    hid_t = jnp.where((jnp.arange(Bn * K) < K)[:, None],
                      hid0[t_flat], hid1[t_flat])
    rel_t = jnp.repeat(rel_emb, K, axis=0)
    sc_t = _score_pallas(hid_t, rel_t, lin_W, lin_b, mlp_W1, mlp_b1,
                         mlp_W2, mlp_b2)
    return sc_t.reshape(Bn, K)
